# baseline (speedup 1.0000x reference)
import functools

import jax
import jax.numpy as jnp
from jax import lax
from jax.experimental import pallas as pl
from jax.experimental.pallas import tpu as pltpu
from jax.experimental.pallas import tpu_sc as plsc

D = 128
H = 8
Z = 16
HZ = H * Z
AUXW = 16
WTAB = HZ + AUXW
L = 16
NC = 2
NS = 16
NW = NC * NS
C = 80


def _axis(name):
    return lax.axis_index(name)


def _barrier():
    plsc.subcore_barrier()


def _ploop(lo, hi, unroll, body):
    plsc.parallel_loop(lo, hi, 1, unroll=unroll)(body)


@functools.partial(jax.jit, static_argnames=())
def _edge_pass(srctab, dsttab, src, dst, ibeta16):
    n = srctab.shape[0]
    e_total = src.shape[0]
    assert e_total % (NW * C) == 0, e_total
    ew = e_total // NW
    chunks = ew // C
    npad = -(-n // (NS * 8)) * NS * 8
    rpt = npad // NS

    mesh = plsc.VectorSubcoreMesh(
        core_axis_name="c", subcore_axis_name="s",
        num_cores=NC, num_subcores=NS)

    @functools.partial(
        pl.kernel,
        out_type=jax.ShapeDtypeStruct((NC, npad, WTAB), jnp.float32),
        mesh=mesh,
        scratch_types=[
            pltpu.VMEM((C,), jnp.int32),
            pltpu.VMEM((C,), jnp.int32),
            pltpu.VMEM((1, C, WTAB), jnp.float32),
            pltpu.VMEM((1, C, HZ), jnp.float32),
            pltpu.VMEM((C, WTAB), jnp.float32),
            pltpu.VMEM((L,), jnp.float32),
            pltpu.VMEM_SHARED((npad, WTAB), jnp.float32),
            pltpu.SemaphoreType.DMA,
            pltpu.SemaphoreType.DMA,
        ],
        compiler_params=pltpu.CompilerParams(needs_layout_passes=False,
                                             use_tc_tiling_on_sc=False),
    )
    def kern(srctab_hbm, dsttab_hbm, src_hbm, dst_hbm, ibeta_hbm, out_hbm,
             sidv0, didv0, srows, drows, msg, ibv, acc,
             sem_s0, sem_d0):
        sidv = (sidv0,)
        didv = (didv0,)
        sem_s = (sem_s0,)
        sem_d = (sem_d0,)
        cid = _axis("c")
        sid = _axis("s")
        wid = sid * NC + cid
        lane = lax.iota(jnp.int32, L)
        zero16 = jnp.zeros((L,), jnp.float32)

        def zrow(i, _):
            for j in range(WTAB // L):
                msg[i, pl.ds(j * L, L)] = zero16
            return 0
        lax.fori_loop(0, C, zrow, 0, unroll=False)
        zbase = sid * rpt
        full = rpt // C
        rem = rpt - full * C
        for j in range(full):
            pltpu.sync_copy(msg, acc.at[pl.ds(zbase + j * C, C)])
        if rem:
            pltpu.sync_copy(msg.at[pl.ds(0, rem)],
                            acc.at[pl.ds(zbase + full * C, rem)])
        pltpu.sync_copy(ibeta_hbm, ibv)
        _barrier()

        ib = ibv[pl.ds(0, L)]
        ebase = wid * ew

        def issue(b, ci):
            base = pl.multiple_of(ebase + ci * C, 8)
            pltpu.sync_copy(src_hbm.at[pl.ds(base, C)], sidv[b])
            pltpu.sync_copy(dst_hbm.at[pl.ds(base, C)], didv[b])
            pltpu.async_copy(srctab_hbm.at[sidv[b]], srows.at[b], sem_s[b])
            pltpu.async_copy(dsttab_hbm.at[didv[b]], drows.at[b], sem_d[b])

        def consume(b, ci):
            pltpu.make_async_copy(srctab_hbm.at[sidv[b]], srows.at[b],
                                  sem_s[b]).wait()
            pltpu.make_async_copy(dsttab_hbm.at[didv[b]], drows.at[b],
                                  sem_d[b]).wait()

            hmask = [jnp.where(lane == h, 1.0, 0.0) for h in range(H)]

            def edge(ei):
                khs = [srows[b, ei, pl.ds(h * L, L)] for h in range(H)]
                parts = []
                for h in range(H):
                    qh = drows[b, ei, pl.ds(h * L, L)]
                    parts.append(jnp.sum(khs[h] * qh) * hmask[h])
                while len(parts) > 1:
                    parts = [parts[i] + parts[i + 1]
                             for i in range(0, len(parts), 2)]
                s = jnp.exp(jnp.clip(parts[0] * ib, -5.0, 5.0))
                aux = srows[b, ei, pl.ds(HZ, L)]
                sb = s * aux
                msg[ei, pl.ds(HZ, L)] = jnp.where(lane < H, sb, 0.0)
                for h in range(H):
                    msg[ei, pl.ds(h * L, L)] = khs[h] * sb[h]
            _ploop(0, C, 2, edge)
            pltpu.sync_copy(msg, acc.at[didv[b]], add=True)

        def chunk(ci, _):
            issue(0, ci)
            consume(0, ci)
            return 0
        lax.fori_loop(0, chunks, chunk, 0, unroll=False)

        _barrier()
        pltpu.sync_copy(acc.at[pl.ds(zbase, rpt)],
                        out_hbm.at[cid, pl.ds(zbase, rpt)])

    return kern(srctab, dsttab, src, dst, ibeta16)


_BLK = 1000


def _tc_project_body(h_ref, wq_ref, wk_ref, qtab_ref, ktab_ref, k144_ref):
    hb = h_ref[...]
    qt = jnp.dot(hb, wq_ref[...], preferred_element_type=jnp.float32)
    kt = jnp.dot(hb, wk_ref[...], preferred_element_type=jnp.float32)
    qtab_ref[...] = qt
    ktab_ref[...] = kt
    cols = lax.broadcasted_iota(jnp.int32, (hb.shape[0], WTAB), 1)
    pad = jnp.pad(kt, ((0, 0), (0, AUXW)))
    k144_ref[...] = pad + jnp.where((cols >= HZ) & (cols < HZ + H), 1.0, 0.0)


def _tc_project(h, wq, wk):
    n = h.shape[0]
    grid = (n // _BLK,)
    return pl.pallas_call(
        _tc_project_body,
        grid=grid,
        in_specs=[
            pl.BlockSpec((_BLK, D), lambda i: (i, 0)),
            pl.BlockSpec((D, HZ), lambda i: (0, 0)),
            pl.BlockSpec((D, HZ), lambda i: (0, 0)),
        ],
        out_specs=[
            pl.BlockSpec((_BLK, HZ), lambda i: (i, 0)),
            pl.BlockSpec((_BLK, HZ), lambda i: (i, 0)),
            pl.BlockSpec((_BLK, WTAB), lambda i: (i, 0)),
        ],
        out_shape=[
            jax.ShapeDtypeStruct((n, HZ), jnp.float32),
            jax.ShapeDtypeStruct((n, HZ), jnp.float32),
            jax.ShapeDtypeStruct((n, WTAB), jnp.float32),
        ],
    )(h, wq, wk)


def _tc_mid_body(g1_ref, qtab_ref, wqt_ref, rep_ref, beta_ref,
                 p2_ref, e1_ref, en_ref):
    g = g1_ref[0] + g1_ref[1]
    g1 = g[:, :HZ]
    zz = g[:, HZ:HZ + H]
    zinv = 1.0 / zz
    zrep = jnp.dot(zinv, rep_ref[...],
                   preferred_element_type=jnp.float32)
    e1_ref[...] = jnp.dot(g1 * zrep, wqt_ref[...],
                          preferred_element_type=jnp.float32)
    cols = lax.broadcasted_iota(jnp.int32, (g.shape[0], WTAB), 1)
    p2 = jnp.pad(qtab_ref[...], ((0, 0), (0, AUXW)))
    zpad = jnp.pad(zinv, ((0, 0), (HZ, AUXW - H)))
    p2_ref[...] = p2 + jnp.where((cols >= HZ) & (cols < HZ + H), zpad, 0.0)
    en_ref[...] = beta_ref[0, 0] * jnp.sum(jnp.log(zz), axis=1, keepdims=True)


def _tc_mid(g1both, qtab, wqt, rep, beta11):
    n = qtab.shape[0]
    grid = (n // _BLK,)
    return pl.pallas_call(
        _tc_mid_body,
        grid=grid,
        in_specs=[
            pl.BlockSpec((NC, _BLK, WTAB), lambda i: (0, i, 0)),
            pl.BlockSpec((_BLK, HZ), lambda i: (i, 0)),
            pl.BlockSpec((HZ, D), lambda i: (0, 0)),
            pl.BlockSpec((H, HZ), lambda i: (0, 0)),
            pl.BlockSpec((1, 1), lambda i: (0, 0), memory_space=pltpu.SMEM),
        ],
        out_specs=[
            pl.BlockSpec((_BLK, WTAB), lambda i: (i, 0)),
            pl.BlockSpec((_BLK, D), lambda i: (i, 0)),
            pl.BlockSpec((_BLK, 1), lambda i: (i, 0)),
        ],
        out_shape=[
            jax.ShapeDtypeStruct((n, WTAB), jnp.float32),
            jax.ShapeDtypeStruct((n, D), jnp.float32),
            jax.ShapeDtypeStruct((n, 1), jnp.float32),
        ],
    )(g1both, qtab, wqt, rep, beta11)


def _tc_final_body(g2_ref, e1_ref, wkt_ref, out_ref):
    g = g2_ref[0] + g2_ref[1]
    g2 = g[:, :HZ]
    out_ref[...] = e1_ref[...] + jnp.dot(g2, wkt_ref[...],
                                         preferred_element_type=jnp.float32)


def _tc_final(g2both, e1, wkt):
    n = e1.shape[0]
    grid = (n // _BLK,)
    return pl.pallas_call(
        _tc_final_body,
        grid=grid,
        in_specs=[
            pl.BlockSpec((NC, _BLK, WTAB), lambda i: (0, i, 0)),
            pl.BlockSpec((_BLK, D), lambda i: (i, 0)),
            pl.BlockSpec((HZ, D), lambda i: (0, 0)),
        ],
        out_specs=pl.BlockSpec((_BLK, D), lambda i: (i, 0)),
        out_shape=jax.ShapeDtypeStruct((n, D), jnp.float32),
    )(g2both, e1, wkt)


def kernel(h, edge_index, Q, K, beta):
    n = h.shape[0]
    src = edge_index[0]
    dst = edge_index[1]
    wq = Q.reshape(D, HZ)
    wk = K.reshape(D, HZ)
    ibeta16 = jnp.full((L,), 1.0, jnp.float32) / beta[0]
    beta11 = beta.reshape(1, 1)
    rep = jnp.kron(jnp.eye(H, dtype=jnp.float32),
                   jnp.ones((1, Z), jnp.float32))

    qtab, ktab, k144 = _tc_project(h, wq, wk)
    g1both = _edge_pass(k144, qtab, src, dst, ibeta16)
    p2, e1, energy = _tc_mid(g1both, qtab, wq.T, rep, beta11)
    g2both = _edge_pass(p2, ktab, src, dst, ibeta16)
    final = _tc_final(g2both, e1, wk.T)
    return (final, energy.reshape(n))


# speedup vs baseline: 1.1204x; 1.1204x over previous
import functools

import jax
import jax.numpy as jnp
from jax import lax
from jax.experimental import pallas as pl
from jax.experimental.pallas import tpu as pltpu
from jax.experimental.pallas import tpu_sc as plsc

D = 128
H = 8
Z = 16
HZ = H * Z
AUXW = 16
WTAB = HZ + AUXW
L = 16
NC = 2
NS = 16
NW = NC * NS
C = 40


def _axis(name):
    return lax.axis_index(name)


def _barrier():
    plsc.subcore_barrier()


def _ploop(lo, hi, unroll, body):
    plsc.parallel_loop(lo, hi, 1, unroll=unroll)(body)


@functools.partial(jax.jit, static_argnames=())
def _edge_pass(srctab, dsttab, src, dst, ibeta16):
    n = srctab.shape[0]
    e_total = src.shape[0]
    assert e_total % (NW * C) == 0, e_total
    ew = e_total // NW
    chunks = ew // C
    npad = -(-n // (NS * 8)) * NS * 8
    rpt = npad // NS

    mesh = plsc.VectorSubcoreMesh(
        core_axis_name="c", subcore_axis_name="s",
        num_cores=NC, num_subcores=NS)

    @functools.partial(
        pl.kernel,
        out_type=jax.ShapeDtypeStruct((NC, npad, WTAB), jnp.float32),
        mesh=mesh,
        scratch_types=[
            pltpu.VMEM((C,), jnp.int32),
            pltpu.VMEM((C,), jnp.int32),
            pltpu.VMEM((C,), jnp.int32),
            pltpu.VMEM((C,), jnp.int32),
            pltpu.VMEM((2, C, WTAB), jnp.float32),
            pltpu.VMEM((2, C, HZ), jnp.float32),
            pltpu.VMEM((C, WTAB), jnp.float32),
            pltpu.VMEM((L,), jnp.float32),
            pltpu.VMEM_SHARED((npad, WTAB), jnp.float32),
            pltpu.SemaphoreType.DMA,
            pltpu.SemaphoreType.DMA,
            pltpu.SemaphoreType.DMA,
            pltpu.SemaphoreType.DMA,
        ],
        compiler_params=pltpu.CompilerParams(needs_layout_passes=False,
                                             use_tc_tiling_on_sc=False),
    )
    def kern(srctab_hbm, dsttab_hbm, src_hbm, dst_hbm, ibeta_hbm, out_hbm,
             sidv0, sidv1, didv0, didv1, srows, drows, msg, ibv, acc,
             sem_s0, sem_s1, sem_d0, sem_d1):
        sidv = (sidv0, sidv1)
        didv = (didv0, didv1)
        sem_s = (sem_s0, sem_s1)
        sem_d = (sem_d0, sem_d1)
        cid = _axis("c")
        sid = _axis("s")
        wid = sid * NC + cid
        lane = lax.iota(jnp.int32, L)
        zero16 = jnp.zeros((L,), jnp.float32)

        def zrow(i, _):
            for j in range(WTAB // L):
                msg[i, pl.ds(j * L, L)] = zero16
            return 0
        lax.fori_loop(0, C, zrow, 0, unroll=False)
        zbase = sid * rpt
        full = rpt // C
        rem = rpt - full * C
        for j in range(full):
            pltpu.sync_copy(msg, acc.at[pl.ds(zbase + j * C, C)])
        if rem:
            pltpu.sync_copy(msg.at[pl.ds(0, rem)],
                            acc.at[pl.ds(zbase + full * C, rem)])
        pltpu.sync_copy(ibeta_hbm, ibv)
        _barrier()

        ib = ibv[pl.ds(0, L)]
        ebase = wid * ew

        def issue(b, ci):
            base = pl.multiple_of(ebase + ci * C, 8)
            pltpu.sync_copy(src_hbm.at[pl.ds(base, C)], sidv[b])
            pltpu.sync_copy(dst_hbm.at[pl.ds(base, C)], didv[b])
            pltpu.async_copy(srctab_hbm.at[sidv[b]], srows.at[b], sem_s[b])
            pltpu.async_copy(dsttab_hbm.at[didv[b]], drows.at[b], sem_d[b])

        def consume(b, ci):
            pltpu.make_async_copy(srctab_hbm.at[sidv[b]], srows.at[b],
                                  sem_s[b]).wait()
            pltpu.make_async_copy(dsttab_hbm.at[didv[b]], drows.at[b],
                                  sem_d[b]).wait()

            hmask = [jnp.where(lane == h, 1.0, 0.0) for h in range(H)]

            def edge(ei):
                khs = [srows[b, ei, pl.ds(h * L, L)] for h in range(H)]
                parts = []
                for h in range(H):
                    qh = drows[b, ei, pl.ds(h * L, L)]
                    parts.append(jnp.sum(khs[h] * qh) * hmask[h])
                while len(parts) > 1:
                    parts = [parts[i] + parts[i + 1]
                             for i in range(0, len(parts), 2)]
                s = jnp.exp(jnp.clip(parts[0] * ib, -5.0, 5.0))
                aux = srows[b, ei, pl.ds(HZ, L)]
                sb = s * aux
                msg[ei, pl.ds(HZ, L)] = jnp.where(lane < H, sb, 0.0)
                for h in range(H):
                    msg[ei, pl.ds(h * L, L)] = khs[h] * sb[h]
            _ploop(0, C, 2, edge)
            pltpu.sync_copy(msg, acc.at[didv[b]], add=True)
            if isinstance(ci, int):
                if ci + 2 < chunks:
                    issue(b, ci + 2)
            else:
                pl.when(ci + 2 < chunks)(lambda: issue(b, ci + 2))

        for b in range(min(2, chunks)):
            issue(b, b)
        pairs = chunks // 2

        def pair(c2, _):
            for b in range(2):
                consume(b, c2 * 2 + b)
            return 0
        lax.fori_loop(0, pairs, pair, 0, unroll=False)
        if chunks % 2:
            consume(0, chunks - 1)

        _barrier()
        pltpu.sync_copy(acc.at[pl.ds(zbase, rpt)],
                        out_hbm.at[cid, pl.ds(zbase, rpt)])

    return kern(srctab, dsttab, src, dst, ibeta16)


_BLK = 1000


def _tc_project_body(h_ref, wq_ref, wk_ref, qtab_ref, ktab_ref, k144_ref):
    hb = h_ref[...]
    qt = jnp.dot(hb, wq_ref[...], preferred_element_type=jnp.float32)
    kt = jnp.dot(hb, wk_ref[...], preferred_element_type=jnp.float32)
    qtab_ref[...] = qt
    ktab_ref[...] = kt
    cols = lax.broadcasted_iota(jnp.int32, (hb.shape[0], WTAB), 1)
    pad = jnp.pad(kt, ((0, 0), (0, AUXW)))
    k144_ref[...] = pad + jnp.where((cols >= HZ) & (cols < HZ + H), 1.0, 0.0)


def _tc_project(h, wq, wk):
    n = h.shape[0]
    grid = (n // _BLK,)
    return pl.pallas_call(
        _tc_project_body,
        grid=grid,
        in_specs=[
            pl.BlockSpec((_BLK, D), lambda i: (i, 0)),
            pl.BlockSpec((D, HZ), lambda i: (0, 0)),
            pl.BlockSpec((D, HZ), lambda i: (0, 0)),
        ],
        out_specs=[
            pl.BlockSpec((_BLK, HZ), lambda i: (i, 0)),
            pl.BlockSpec((_BLK, HZ), lambda i: (i, 0)),
            pl.BlockSpec((_BLK, WTAB), lambda i: (i, 0)),
        ],
        out_shape=[
            jax.ShapeDtypeStruct((n, HZ), jnp.float32),
            jax.ShapeDtypeStruct((n, HZ), jnp.float32),
            jax.ShapeDtypeStruct((n, WTAB), jnp.float32),
        ],
    )(h, wq, wk)


def _tc_mid_body(g1_ref, qtab_ref, wqt_ref, rep_ref, beta_ref,
                 p2_ref, e1_ref, en_ref):
    g = g1_ref[0] + g1_ref[1]
    g1 = g[:, :HZ]
    zz = g[:, HZ:HZ + H]
    zinv = 1.0 / zz
    zrep = jnp.dot(zinv, rep_ref[...],
                   preferred_element_type=jnp.float32)
    e1_ref[...] = jnp.dot(g1 * zrep, wqt_ref[...],
                          preferred_element_type=jnp.float32)
    cols = lax.broadcasted_iota(jnp.int32, (g.shape[0], WTAB), 1)
    p2 = jnp.pad(qtab_ref[...], ((0, 0), (0, AUXW)))
    zpad = jnp.pad(zinv, ((0, 0), (HZ, AUXW - H)))
    p2_ref[...] = p2 + jnp.where((cols >= HZ) & (cols < HZ + H), zpad, 0.0)
    en_ref[...] = beta_ref[0, 0] * jnp.sum(jnp.log(zz), axis=1, keepdims=True)


def _tc_mid(g1both, qtab, wqt, rep, beta11):
    n = qtab.shape[0]
    grid = (n // _BLK,)
    return pl.pallas_call(
        _tc_mid_body,
        grid=grid,
        in_specs=[
            pl.BlockSpec((NC, _BLK, WTAB), lambda i: (0, i, 0)),
            pl.BlockSpec((_BLK, HZ), lambda i: (i, 0)),
            pl.BlockSpec((HZ, D), lambda i: (0, 0)),
            pl.BlockSpec((H, HZ), lambda i: (0, 0)),
            pl.BlockSpec((1, 1), lambda i: (0, 0), memory_space=pltpu.SMEM),
        ],
        out_specs=[
            pl.BlockSpec((_BLK, WTAB), lambda i: (i, 0)),
            pl.BlockSpec((_BLK, D), lambda i: (i, 0)),
            pl.BlockSpec((_BLK, 1), lambda i: (i, 0)),
        ],
        out_shape=[
            jax.ShapeDtypeStruct((n, WTAB), jnp.float32),
            jax.ShapeDtypeStruct((n, D), jnp.float32),
            jax.ShapeDtypeStruct((n, 1), jnp.float32),
        ],
    )(g1both, qtab, wqt, rep, beta11)


def _tc_final_body(g2_ref, e1_ref, wkt_ref, out_ref):
    g = g2_ref[0] + g2_ref[1]
    g2 = g[:, :HZ]
    out_ref[...] = e1_ref[...] + jnp.dot(g2, wkt_ref[...],
                                         preferred_element_type=jnp.float32)


def _tc_final(g2both, e1, wkt):
    n = e1.shape[0]
    grid = (n // _BLK,)
    return pl.pallas_call(
        _tc_final_body,
        grid=grid,
        in_specs=[
            pl.BlockSpec((NC, _BLK, WTAB), lambda i: (0, i, 0)),
            pl.BlockSpec((_BLK, D), lambda i: (i, 0)),
            pl.BlockSpec((HZ, D), lambda i: (0, 0)),
        ],
        out_specs=pl.BlockSpec((_BLK, D), lambda i: (i, 0)),
        out_shape=jax.ShapeDtypeStruct((n, D), jnp.float32),
    )(g2both, e1, wkt)


def kernel(h, edge_index, Q, K, beta):
    n = h.shape[0]
    src = edge_index[0]
    dst = edge_index[1]
    wq = Q.reshape(D, HZ)
    wk = K.reshape(D, HZ)
    ibeta16 = jnp.full((L,), 1.0, jnp.float32) / beta[0]
    beta11 = beta.reshape(1, 1)
    rep = jnp.kron(jnp.eye(H, dtype=jnp.float32),
                   jnp.ones((1, Z), jnp.float32))

    qtab, ktab, k144 = _tc_project(h, wq, wk)
    g1both = _edge_pass(k144, qtab, src, dst, ibeta16)
    p2, e1, energy = _tc_mid(g1both, qtab, wq.T, rep, beta11)
    g2both = _edge_pass(p2, ktab, src, dst, ibeta16)
    final = _tc_final(g2both, e1, wk.T)
    return (final, energy.reshape(n))
